# baseline (speedup 1.0000x reference)
import functools

import jax
import jax.numpy as jnp
from jax import lax
from jax.experimental import pallas as pl
from jax.experimental.pallas import tpu as pltpu
from jax.experimental.pallas import tpu_sc as plsc

_NC = 2
_NS = 16
_NW = _NC * _NS
_D = 128
_CHUNK = 200


def _make_sc_gather(n_rows: int):
    rows_per_w = n_rows // _NW
    n_chunks = rows_per_w // _CHUNK
    n_quads = n_chunks // 4
    mesh = plsc.VectorSubcoreMesh(core_axis_name="c", subcore_axis_name="s")

    @functools.partial(
        pl.kernel,
        out_type=jax.ShapeDtypeStruct((n_rows, _D), jnp.float32),
        mesh=mesh,
        scratch_types=[
            pltpu.VMEM((rows_per_w,), jnp.int32),
            pltpu.VMEM((_CHUNK, _D), jnp.float32),
            pltpu.VMEM((_CHUNK, _D), jnp.float32),
            pltpu.VMEM((_CHUNK, _D), jnp.float32),
            pltpu.VMEM((_CHUNK, _D), jnp.float32),
            pltpu.SemaphoreType.DMA,
            pltpu.SemaphoreType.DMA,
            pltpu.SemaphoreType.DMA,
            pltpu.SemaphoreType.DMA,
            pltpu.SemaphoreType.DMA,
            pltpu.SemaphoreType.DMA,
            pltpu.SemaphoreType.DMA,
            pltpu.SemaphoreType.DMA,
        ],
    )
    def gather_kernel(ids_hbm, table_hbm, out_hbm, idx_v, b0, b1, b2, b3,
                      gs0, gs1, gs2, gs3, ws0, ws1, ws2, ws3):
        wid = lax.axis_index("s") * _NC + lax.axis_index("c")
        base = wid * rows_per_w
        pltpu.sync_copy(ids_hbm.at[pl.ds(base, rows_per_w)], idx_v)

        bufs = (b0, b1, b2, b3)
        gsems = (gs0, gs1, gs2, gs3)
        wsems = (ws0, ws1, ws2, ws3)

        def gather_start(g, j):
            pltpu.async_copy(
                table_hbm.at[idx_v.at[pl.ds(g * _CHUNK, _CHUNK)]],
                bufs[j], gsems[j])

        def gather_wait(j):
            pltpu.make_async_copy(
                table_hbm.at[idx_v.at[pl.ds(0, _CHUNK)]], bufs[j],
                gsems[j]).wait()

        def write_start(g, j):
            pltpu.async_copy(
                bufs[j], out_hbm.at[pl.ds(base + g * _CHUNK, _CHUNK)],
                wsems[j])

        def write_wait(j):
            pltpu.make_async_copy(
                bufs[j], out_hbm.at[pl.ds(base, _CHUNK)], wsems[j]).wait()

        gather_start(0, 0)
        gather_start(1, 1)

        def quad_body(q, carry):
            for j in range(4):
                g = 4 * q + j
                jn = (j + 2) % 4

                @pl.when(g >= 2)
                def _():
                    write_wait(jn)

                @pl.when(g + 2 < n_chunks)
                def _():
                    gather_start(g + 2, jn)

                gather_wait(j)
                write_start(g, j)
            return carry

        lax.fori_loop(0, n_quads, quad_body, 0)
        write_wait((n_chunks - 2) % 4)
        write_wait((n_chunks - 1) % 4)

    return gather_kernel


_ROWS_BLK = 12800
_EPS = 1e-12


def _ln_body(x_ref, pos_ref, tt_ref, gamma_ref, beta_ref, o_ref):
    x = x_ref[...].reshape(_ROWS_BLK // 200, 200, _D)
    bias = pos_ref[...] + tt_ref[0][None, :]
    h = x + bias[None]
    mean = jnp.mean(h, axis=-1, keepdims=True)
    c = h - mean
    var = jnp.mean(c * c, axis=-1, keepdims=True)
    normed = c * lax.rsqrt(var + _EPS)
    out = normed * gamma_ref[0][None, None, :] + beta_ref[0][None, None, :]
    o_ref[...] = out.reshape(_ROWS_BLK, _D)


def _layernorm(gathered, W_pos_t, W_tt, gamma2d, beta2d):
    n_rows = gathered.shape[0]
    grid = (n_rows // _ROWS_BLK,)
    return pl.pallas_call(
        _ln_body,
        grid=grid,
        in_specs=[
            pl.BlockSpec((_ROWS_BLK, _D), lambda i: (i, 0)),
            pl.BlockSpec((200, _D), lambda i: (0, 0)),
            pl.BlockSpec((2, _D), lambda i: (0, 0)),
            pl.BlockSpec((1, _D), lambda i: (0, 0)),
            pl.BlockSpec((1, _D), lambda i: (0, 0)),
        ],
        out_specs=pl.BlockSpec((_ROWS_BLK, _D), lambda i: (i, 0)),
        out_shape=jax.ShapeDtypeStruct((n_rows, _D), jnp.float32),
    )(gathered, W_pos_t, W_tt, gamma2d, beta2d)


_N_PIPE = 2


def kernel(input_ids, W_word, W_pos, W_tt, gamma, beta):
    B, T = input_ids.shape
    ids_flat = input_ids.reshape(-1).astype(jnp.int32)
    rpc = (B * T) // _N_PIPE
    gather_fn = _make_sc_gather(rpc)
    pos_t = W_pos[:T]
    gamma2d = gamma.reshape(1, _D)
    beta2d = beta.reshape(1, _D)
    outs = []
    for k in range(_N_PIPE):
        g_k = gather_fn(lax.dynamic_slice(ids_flat, (k * rpc,), (rpc,)), W_word)
        outs.append(_layernorm(g_k, pos_t, W_tt, gamma2d, beta2d))
    return jnp.concatenate(outs).reshape(B, T, _D)


# speedup vs baseline: 1.3323x; 1.3323x over previous
import functools

import jax
import jax.numpy as jnp
from jax import lax
from jax.experimental import pallas as pl
from jax.experimental.pallas import tpu as pltpu
from jax.experimental.pallas import tpu_sc as plsc

_NC = 2
_NS = 16
_NW = _NC * _NS
_D = 128
_CHUNK = 200


def _make_sc_gather(n_rows: int):
    rows_per_w = n_rows // _NW
    n_chunks = rows_per_w // _CHUNK
    n_quads = n_chunks // 4
    mesh = plsc.VectorSubcoreMesh(core_axis_name="c", subcore_axis_name="s")

    @functools.partial(
        pl.kernel,
        out_type=jax.ShapeDtypeStruct((n_rows, _D), jnp.float32),
        mesh=mesh,
        scratch_types=[
            pltpu.VMEM((rows_per_w,), jnp.int32),
            pltpu.VMEM((_CHUNK, _D), jnp.float32),
            pltpu.VMEM((_CHUNK, _D), jnp.float32),
            pltpu.VMEM((_CHUNK, _D), jnp.float32),
            pltpu.VMEM((_CHUNK, _D), jnp.float32),
            pltpu.SemaphoreType.DMA,
            pltpu.SemaphoreType.DMA,
            pltpu.SemaphoreType.DMA,
            pltpu.SemaphoreType.DMA,
            pltpu.SemaphoreType.DMA,
            pltpu.SemaphoreType.DMA,
            pltpu.SemaphoreType.DMA,
            pltpu.SemaphoreType.DMA,
        ],
    )
    def gather_kernel(ids_hbm, table_hbm, out_hbm, idx_v, b0, b1, b2, b3,
                      gs0, gs1, gs2, gs3, ws0, ws1, ws2, ws3):
        wid = lax.axis_index("s") * _NC + lax.axis_index("c")
        base = wid * rows_per_w
        pltpu.sync_copy(ids_hbm.at[pl.ds(base, rows_per_w)], idx_v)

        bufs = (b0, b1, b2, b3)
        gsems = (gs0, gs1, gs2, gs3)
        wsems = (ws0, ws1, ws2, ws3)

        def gather_start(g, j):
            pltpu.async_copy(
                table_hbm.at[idx_v.at[pl.ds(g * _CHUNK, _CHUNK)]],
                bufs[j], gsems[j])

        def gather_wait(j):
            pltpu.make_async_copy(
                table_hbm.at[idx_v.at[pl.ds(0, _CHUNK)]], bufs[j],
                gsems[j]).wait()

        def write_start(g, j):
            pltpu.async_copy(
                bufs[j], out_hbm.at[pl.ds(base + g * _CHUNK, _CHUNK)],
                wsems[j])

        def write_wait(j):
            pltpu.make_async_copy(
                bufs[j], out_hbm.at[pl.ds(base, _CHUNK)], wsems[j]).wait()

        gather_start(0, 0)
        gather_start(1, 1)

        def quad_body(q, carry):
            for j in range(4):
                g = 4 * q + j
                jn = (j + 2) % 4

                @pl.when(g >= 2)
                def _():
                    write_wait(jn)

                @pl.when(g + 2 < n_chunks)
                def _():
                    gather_start(g + 2, jn)

                gather_wait(j)
                write_start(g, j)
            return carry

        lax.fori_loop(0, n_quads, quad_body, 0)
        write_wait((n_chunks - 2) % 4)
        write_wait((n_chunks - 1) % 4)

    return gather_kernel


_ROWS_BLK = 12800
_SUB = 1600
_EPS = 1e-12


def _ln_body(x_ref, pos_ref, tt_ref, gamma_ref, beta_ref, o_ref):
    bias = pos_ref[...] + tt_ref[0][None, :]
    gamma_row = gamma_ref[0][None, None, :]
    beta_row = beta_ref[0][None, None, :]

    def sub_tile(i, carry):
        x = x_ref[pl.ds(i * _SUB, _SUB), :].reshape(_SUB // 200, 200, _D)
        h = x + bias[None]
        mean = jnp.mean(h, axis=-1, keepdims=True)
        c = h - mean
        var = jnp.mean(c * c, axis=-1, keepdims=True)
        normed = c * lax.rsqrt(var + _EPS)
        out = normed * gamma_row + beta_row
        o_ref[pl.ds(i * _SUB, _SUB), :] = out.reshape(_SUB, _D)
        return carry

    lax.fori_loop(0, _ROWS_BLK // _SUB, sub_tile, 0)


def _layernorm(gathered, W_pos_t, W_tt, gamma2d, beta2d):
    n_rows = gathered.shape[0]
    grid = (n_rows // _ROWS_BLK,)
    return pl.pallas_call(
        _ln_body,
        grid=grid,
        in_specs=[
            pl.BlockSpec((_ROWS_BLK, _D), lambda i: (i, 0)),
            pl.BlockSpec((200, _D), lambda i: (0, 0)),
            pl.BlockSpec((2, _D), lambda i: (0, 0)),
            pl.BlockSpec((1, _D), lambda i: (0, 0)),
            pl.BlockSpec((1, _D), lambda i: (0, 0)),
        ],
        out_specs=pl.BlockSpec((_ROWS_BLK, _D), lambda i: (i, 0)),
        out_shape=jax.ShapeDtypeStruct((n_rows, _D), jnp.float32),
    )(gathered, W_pos_t, W_tt, gamma2d, beta2d)


def kernel(input_ids, W_word, W_pos, W_tt, gamma, beta):
    B, T = input_ids.shape
    ids_flat = input_ids.reshape(-1).astype(jnp.int32)
    gathered = _make_sc_gather(B * T)(ids_flat, W_word)
    out = _layernorm(
        gathered,
        W_pos[:T],
        W_tt,
        gamma.reshape(1, _D),
        beta.reshape(1, _D),
    )
    return out.reshape(B, T, _D)


# speedup vs baseline: 1.3534x; 1.0158x over previous
import functools

import jax
import jax.numpy as jnp
from jax import lax
from jax.experimental import pallas as pl
from jax.experimental.pallas import tpu as pltpu
from jax.experimental.pallas import tpu_sc as plsc

_NC = 2
_NS = 16
_NW = _NC * _NS
_D = 128
_CHUNK = 200


def _make_sc_gather(n_rows: int):
    rows_per_w = n_rows // _NW
    n_chunks = rows_per_w // _CHUNK
    n_quads = n_chunks // 4
    mesh = plsc.VectorSubcoreMesh(core_axis_name="c", subcore_axis_name="s")

    @functools.partial(
        pl.kernel,
        out_type=jax.ShapeDtypeStruct((n_rows, _D), jnp.float32),
        mesh=mesh,
        scratch_types=[
            pltpu.VMEM((rows_per_w,), jnp.int32),
            pltpu.VMEM((_CHUNK, _D), jnp.float32),
            pltpu.VMEM((_CHUNK, _D), jnp.float32),
            pltpu.VMEM((_CHUNK, _D), jnp.float32),
            pltpu.VMEM((_CHUNK, _D), jnp.float32),
            pltpu.SemaphoreType.DMA,
            pltpu.SemaphoreType.DMA,
            pltpu.SemaphoreType.DMA,
            pltpu.SemaphoreType.DMA,
            pltpu.SemaphoreType.DMA,
            pltpu.SemaphoreType.DMA,
            pltpu.SemaphoreType.DMA,
            pltpu.SemaphoreType.DMA,
        ],
    )
    def gather_kernel(ids_hbm, table_hbm, out_hbm, idx_v, b0, b1, b2, b3,
                      gs0, gs1, gs2, gs3, ws0, ws1, ws2, ws3):
        wid = lax.axis_index("s") * _NC + lax.axis_index("c")
        base = wid * rows_per_w
        pltpu.sync_copy(ids_hbm.at[pl.ds(base, rows_per_w)], idx_v)

        bufs = (b0, b1, b2, b3)
        gsems = (gs0, gs1, gs2, gs3)
        wsems = (ws0, ws1, ws2, ws3)

        def gather_start(g, j):
            pltpu.async_copy(
                table_hbm.at[idx_v.at[pl.ds(g * _CHUNK, _CHUNK)]],
                bufs[j], gsems[j])

        def gather_wait(j):
            pltpu.make_async_copy(
                table_hbm.at[idx_v.at[pl.ds(0, _CHUNK)]], bufs[j],
                gsems[j]).wait()

        def write_start(g, j):
            pltpu.async_copy(
                bufs[j], out_hbm.at[pl.ds(base + g * _CHUNK, _CHUNK)],
                wsems[j])

        def write_wait(j):
            pltpu.make_async_copy(
                bufs[j], out_hbm.at[pl.ds(base, _CHUNK)], wsems[j]).wait()

        gather_start(0, 0)
        gather_start(1, 1)

        def quad_body(q, carry):
            for j in range(4):
                g = 4 * q + j
                jn = (j + 2) % 4

                @pl.when(g >= 2)
                def _():
                    write_wait(jn)

                @pl.when(g + 2 < n_chunks)
                def _():
                    gather_start(g + 2, jn)

                gather_wait(j)
                write_start(g, j)
            return carry

        lax.fori_loop(0, n_quads, quad_body, 0)
        write_wait((n_chunks - 2) % 4)
        write_wait((n_chunks - 1) % 4)

    return gather_kernel


_ROWS_BLK = 12800
_SUB = 3200
_EPS = 1e-12


def _ln_body(x_ref, pos_ref, tt_ref, gamma_ref, beta_ref, o_ref):
    bias = pos_ref[...] + tt_ref[0][None, :]
    gamma_row = gamma_ref[0][None, None, :]
    beta_row = beta_ref[0][None, None, :]

    def sub_tile(i, carry):
        x = x_ref[pl.ds(i * _SUB, _SUB), :].reshape(_SUB // 200, 200, _D)
        h = x + bias[None]
        mean = jnp.mean(h, axis=-1, keepdims=True)
        c = h - mean
        var = jnp.mean(c * c, axis=-1, keepdims=True)
        normed = c * lax.rsqrt(var + _EPS)
        out = normed * gamma_row + beta_row
        o_ref[pl.ds(i * _SUB, _SUB), :] = out.reshape(_SUB, _D)
        return carry

    lax.fori_loop(0, _ROWS_BLK // _SUB, sub_tile, 0)


def _layernorm(gathered, W_pos_t, W_tt, gamma2d, beta2d):
    n_rows = gathered.shape[0]
    grid = (n_rows // _ROWS_BLK,)
    return pl.pallas_call(
        _ln_body,
        grid=grid,
        in_specs=[
            pl.BlockSpec((_ROWS_BLK, _D), lambda i: (i, 0)),
            pl.BlockSpec((200, _D), lambda i: (0, 0)),
            pl.BlockSpec((2, _D), lambda i: (0, 0)),
            pl.BlockSpec((1, _D), lambda i: (0, 0)),
            pl.BlockSpec((1, _D), lambda i: (0, 0)),
        ],
        out_specs=pl.BlockSpec((_ROWS_BLK, _D), lambda i: (i, 0)),
        out_shape=jax.ShapeDtypeStruct((n_rows, _D), jnp.float32),
    )(gathered, W_pos_t, W_tt, gamma2d, beta2d)


def kernel(input_ids, W_word, W_pos, W_tt, gamma, beta):
    B, T = input_ids.shape
    ids_flat = input_ids.reshape(-1).astype(jnp.int32)
    gathered = _make_sc_gather(B * T)(ids_flat, W_word)
    out = _layernorm(
        gathered,
        W_pos[:T],
        W_tt,
        gamma.reshape(1, _D),
        beta.reshape(1, _D),
    )
    return out.reshape(B, T, _D)


# speedup vs baseline: 1.3592x; 1.0043x over previous
import functools

import jax
import jax.numpy as jnp
from jax import lax
from jax.experimental import pallas as pl
from jax.experimental.pallas import tpu as pltpu
from jax.experimental.pallas import tpu_sc as plsc

_NC = 2
_NS = 16
_NW = _NC * _NS
_D = 128
_CHUNK = 200


def _make_sc_gather(n_rows: int):
    rows_per_w = n_rows // _NW
    n_chunks = rows_per_w // _CHUNK
    n_quads = n_chunks // 4
    mesh = plsc.VectorSubcoreMesh(core_axis_name="c", subcore_axis_name="s")

    @functools.partial(
        pl.kernel,
        out_type=jax.ShapeDtypeStruct((n_rows, _D), jnp.float32),
        mesh=mesh,
        scratch_types=[
            pltpu.VMEM((rows_per_w,), jnp.int32),
            pltpu.VMEM((_CHUNK, _D), jnp.float32),
            pltpu.VMEM((_CHUNK, _D), jnp.float32),
            pltpu.VMEM((_CHUNK, _D), jnp.float32),
            pltpu.VMEM((_CHUNK, _D), jnp.float32),
            pltpu.SemaphoreType.DMA,
            pltpu.SemaphoreType.DMA,
            pltpu.SemaphoreType.DMA,
            pltpu.SemaphoreType.DMA,
            pltpu.SemaphoreType.DMA,
            pltpu.SemaphoreType.DMA,
            pltpu.SemaphoreType.DMA,
            pltpu.SemaphoreType.DMA,
        ],
    )
    def gather_kernel(ids_hbm, table_hbm, out_hbm, idx_v, b0, b1, b2, b3,
                      gs0, gs1, gs2, gs3, ws0, ws1, ws2, ws3):
        wid = lax.axis_index("s") * _NC + lax.axis_index("c")
        base = wid * rows_per_w
        pltpu.sync_copy(ids_hbm.at[pl.ds(base, rows_per_w)], idx_v)

        bufs = (b0, b1, b2, b3)
        gsems = (gs0, gs1, gs2, gs3)
        wsems = (ws0, ws1, ws2, ws3)

        def gather_start(g, j):
            pltpu.async_copy(
                table_hbm.at[idx_v.at[pl.ds(g * _CHUNK, _CHUNK)]],
                bufs[j], gsems[j])

        def gather_wait(j):
            pltpu.make_async_copy(
                table_hbm.at[idx_v.at[pl.ds(0, _CHUNK)]], bufs[j],
                gsems[j]).wait()

        def write_start(g, j):
            pltpu.async_copy(
                bufs[j], out_hbm.at[pl.ds(base + g * _CHUNK, _CHUNK)],
                wsems[j])

        def write_wait(j):
            pltpu.make_async_copy(
                bufs[j], out_hbm.at[pl.ds(base, _CHUNK)], wsems[j]).wait()

        gather_start(0, 0)
        gather_start(1, 1)

        def quad_body(q, carry):
            for j in range(4):
                g = 4 * q + j
                jn = (j + 2) % 4

                @pl.when(g >= 2)
                def _():
                    write_wait(jn)

                @pl.when(g + 2 < n_chunks)
                def _():
                    gather_start(g + 2, jn)

                gather_wait(j)
                write_start(g, j)
            return carry

        lax.fori_loop(0, n_quads, quad_body, 0)
        write_wait((n_chunks - 2) % 4)
        write_wait((n_chunks - 1) % 4)

    return gather_kernel


_ROWS_BLK = 12800
_SUB = 6400
_EPS = 1e-12


def _ln_body(x_ref, pos_ref, tt_ref, gamma_ref, beta_ref, o_ref):
    bias = pos_ref[...] + tt_ref[0][None, :]
    gamma_row = gamma_ref[0][None, None, :]
    beta_row = beta_ref[0][None, None, :]

    def sub_tile(i, carry):
        x = x_ref[pl.ds(i * _SUB, _SUB), :].reshape(_SUB // 200, 200, _D)
        h = x + bias[None]
        mean = jnp.mean(h, axis=-1, keepdims=True)
        c = h - mean
        var = jnp.mean(c * c, axis=-1, keepdims=True)
        normed = c * lax.rsqrt(var + _EPS)
        out = normed * gamma_row + beta_row
        o_ref[pl.ds(i * _SUB, _SUB), :] = out.reshape(_SUB, _D)
        return carry

    lax.fori_loop(0, _ROWS_BLK // _SUB, sub_tile, 0)


def _layernorm(gathered, W_pos_t, W_tt, gamma2d, beta2d):
    n_rows = gathered.shape[0]
    grid = (n_rows // _ROWS_BLK,)
    return pl.pallas_call(
        _ln_body,
        grid=grid,
        in_specs=[
            pl.BlockSpec((_ROWS_BLK, _D), lambda i: (i, 0)),
            pl.BlockSpec((200, _D), lambda i: (0, 0)),
            pl.BlockSpec((2, _D), lambda i: (0, 0)),
            pl.BlockSpec((1, _D), lambda i: (0, 0)),
            pl.BlockSpec((1, _D), lambda i: (0, 0)),
        ],
        out_specs=pl.BlockSpec((_ROWS_BLK, _D), lambda i: (i, 0)),
        out_shape=jax.ShapeDtypeStruct((n_rows, _D), jnp.float32),
    )(gathered, W_pos_t, W_tt, gamma2d, beta2d)


def kernel(input_ids, W_word, W_pos, W_tt, gamma, beta):
    B, T = input_ids.shape
    ids_flat = input_ids.reshape(-1).astype(jnp.int32)
    gathered = _make_sc_gather(B * T)(ids_flat, W_word)
    out = _layernorm(
        gathered,
        W_pos[:T],
        W_tt,
        gamma.reshape(1, _D),
        beta.reshape(1, _D),
    )
    return out.reshape(B, T, _D)
